# Initial kernel scaffold; baseline (speedup 1.0000x reference)
import jax
import jax.numpy as jnp
from jax.experimental import pallas as pl


def kernel(category, table, W, b):
    raise NotImplementedError("write your pallas kernel here")


# speedup vs baseline: 5.7355x; 5.7355x over previous
import functools

import jax
import jax.numpy as jnp
from jax import lax
from jax.experimental import pallas as pl
from jax.experimental.pallas import tpu as pltpu
from jax.experimental.pallas import tpu_sc as plsc

_NC, _NS = 2, 16
_NW = _NC * _NS

_STREAM = 128
_GROUP = 512
_SPG = _GROUP // _STREAM


def _transform_table(table, W, b2d):
    V, E = table.shape
    O = W.shape[0]
    BLK = 4000

    def body(t_ref, w_ref, b_ref, o_ref):
        x = t_ref[...]
        row = lax.broadcasted_iota(jnp.int32, x.shape, 0)
        x = jnp.where((row == 0) & (pl.program_id(0) == 0), 0.0, x)
        y = lax.dot_general(x, w_ref[...], (((1,), (1,)), ((), ())),
                            preferred_element_type=jnp.float32)
        o_ref[...] = jnp.maximum(y + b_ref[...], 0.0)

    return pl.pallas_call(
        body,
        grid=(V // BLK,),
        in_specs=[
            pl.BlockSpec((BLK, E), lambda i: (i, 0)),
            pl.BlockSpec((O, E), lambda i: (0, 0)),
            pl.BlockSpec((1, O), lambda i: (0, 0)),
        ],
        out_specs=pl.BlockSpec((BLK, O), lambda i: (i, 0)),
        out_shape=jax.ShapeDtypeStruct((V, O), jnp.float32),
    )(table, W, b2d)


def _make_gather(B, D):
    assert B % (_NW * _GROUP) == 0
    b_per_w = B // _NW
    n_groups = b_per_w // _GROUP
    mesh = plsc.VectorSubcoreMesh(core_axis_name="c", subcore_axis_name="s")

    @functools.partial(
        pl.kernel,
        mesh=mesh,
        out_type=jax.ShapeDtypeStruct((B, D), jnp.float32),
        compiler_params=pltpu.CompilerParams(use_tc_tiling_on_sc=False),
        scratch_types=[
            pltpu.VMEM((b_per_w,), jnp.int32),
            pltpu.VMEM((2, _GROUP, D), jnp.float32),
            pltpu.SemaphoreType.DMA,
            pltpu.SemaphoreType.DMA,
        ],
    )
    def gather_kernel(t2_hbm, idx_hbm, out_hbm, idx_v, rows_v, sem0, sem1):
        wid = lax.axis_index("s") * _NC + lax.axis_index("c")
        base = wid * b_per_w
        pltpu.sync_copy(idx_hbm.at[pl.ds(base, b_per_w)], idx_v)

        sems = (sem0, sem1)

        def fire(g, slot):
            for j in range(_SPG):
                off = g * _GROUP + j * _STREAM
                pltpu.async_copy(
                    t2_hbm.at[idx_v.at[pl.ds(off, _STREAM)]],
                    rows_v.at[slot, pl.ds(j * _STREAM, _STREAM)],
                    sems[slot],
                )

        def drain(slot):
            pltpu.make_async_copy(
                t2_hbm.at[pl.ds(0, _GROUP)],
                rows_v.at[slot],
                sems[slot],
            ).wait()

        fire(0, 0)
        def body(i, carry):
            g0 = i * 2
            fire(g0 + 1, 1)
            drain(0)
            pltpu.sync_copy(rows_v.at[0],
                            out_hbm.at[pl.ds(base + g0 * _GROUP, _GROUP)])
            @pl.when(i + 1 < n_groups // 2)
            def _():
                fire(g0 + 2, 0)
            drain(1)
            pltpu.sync_copy(rows_v.at[1],
                            out_hbm.at[pl.ds(base + (g0 + 1) * _GROUP, _GROUP)])
            return carry

        lax.fori_loop(0, n_groups // 2, body, 0)

    return gather_kernel


def kernel(category, table, W, b):
    B, H = category.shape
    D = W.shape[0]
    t2 = _transform_table(table, W, b.reshape(1, -1))
    idx = category.reshape(-1).astype(jnp.int32)
    out = _make_gather(B * H, D)(t2, idx)
    return out.reshape(B, H, D)
